# baseline (speedup 1.0000x reference)
import functools

import jax
import jax.numpy as jnp
from jax import lax
from jax.experimental import pallas as pl
from jax.experimental.pallas import tpu as pltpu
from jax.experimental.pallas import tpu_sc as plsc

N = 10000
E = 320000
HIDDEN = 256
FH = 128
ACCW = FH
NUM_GRAPHS = 64
OUT_DIM = 64
BN = 1000
GRID = N // BN

NSC = 2
NSUB = 16
ET = 332800
EB = 2080
NB = 10
ES = NB * EB
G = 80
NQ = EB // G
NH = 5000
NACC = 5120
RW = 320
NDT = 10016


def _gelu_exact(x):
    return 0.5 * x * (1.0 + jax.lax.erf(x * jnp.float32(0.7071067811865476)))


def _mm_att_body(first, t0_ref, t1_ref, b_ref, W_ref, as_ref, ad_ref,
                 H_ref, asrc_ref, adst_ref, cmax_ref):
    i = pl.program_id(0)
    t = jnp.concatenate([t0_ref[...], t1_ref[...]], axis=1)
    if not first:
        t = _gelu_exact(t + b_ref[...])
    H = jax.lax.dot_general(t, W_ref[...], (((1,), (0,)), ((), ())),
                            preferred_element_type=jnp.float32,
                            precision=jax.lax.Precision.HIGHEST)
    H_ref[0] = H[:, :FH]
    H_ref[1] = H[:, FH:]
    asrc = jnp.sum(H * as_ref[...], axis=1, keepdims=True)
    adst = jnp.sum(H * ad_ref[...], axis=1, keepdims=True)
    asrc_ref[...] = asrc
    adst_ref[...] = adst
    m = jnp.concatenate([jnp.max(asrc, axis=0), jnp.max(adst, axis=0)])[None, :]

    @pl.when(i == 0)
    def _():
        cmax_ref[...] = m

    @pl.when(i > 0)
    def _():
        cmax_ref[...] = jnp.maximum(cmax_ref[...], m)


def _mm_att(t0, t1, b, W, att_s, att_d, first):
    body = functools.partial(_mm_att_body, first)
    return pl.pallas_call(
        body,
        grid=(GRID,),
        in_specs=[
            pl.BlockSpec((BN, FH), lambda i: (i, 0)),
            pl.BlockSpec((BN, FH), lambda i: (i, 0)),
            pl.BlockSpec((1, HIDDEN), lambda i: (0, 0)),
            pl.BlockSpec((HIDDEN, HIDDEN), lambda i: (0, 0)),
            pl.BlockSpec((1, HIDDEN), lambda i: (0, 0)),
            pl.BlockSpec((1, HIDDEN), lambda i: (0, 0)),
        ],
        out_specs=[
            pl.BlockSpec((NSC, BN, FH), lambda i: (0, i, 0)),
            pl.BlockSpec((BN, 1), lambda i: (i, 0)),
            pl.BlockSpec((BN, 1), lambda i: (i, 0)),
            pl.BlockSpec((1, 2), lambda i: (0, 0)),
        ],
        out_shape=[
            jax.ShapeDtypeStruct((NSC, N, FH), jnp.float32),
            jax.ShapeDtypeStruct((N, 1), jnp.float32),
            jax.ShapeDtypeStruct((N, 1), jnp.float32),
            jax.ShapeDtypeStruct((1, 2), jnp.float32),
        ],
    )(t0, t1, b[None, :], W, att_s[None, :], att_d[None, :])


def _pool_final_body(t0_ref, t1_ref, b_ref, batch_ref, Wf_ref, bf_ref,
                     out_ref, acc_ref):
    i = pl.program_id(0)
    t = jnp.concatenate([t0_ref[...], t1_ref[...]], axis=1)
    t = _gelu_exact(t + b_ref[...])
    g = batch_ref[...]
    onehot = (g == jax.lax.broadcasted_iota(jnp.int32, (1, NUM_GRAPHS), 1)
              ).astype(jnp.float32)
    part = jax.lax.dot_general(onehot, t, (((0,), (0,)), ((), ())),
                               preferred_element_type=jnp.float32,
                               precision=jax.lax.Precision.HIGHEST)

    @pl.when(i == 0)
    def _():
        acc_ref[...] = jnp.zeros_like(acc_ref)

    acc_ref[...] += part

    @pl.when(i == GRID - 1)
    def _():
        o = jax.lax.dot_general(acc_ref[...], Wf_ref[...], (((1,), (0,)), ((), ())),
                                preferred_element_type=jnp.float32,
                                precision=jax.lax.Precision.HIGHEST) + bf_ref[...]
        out_ref[...] = jnp.where(o >= 0, o, 0.01 * o)


def _pool_final(t0, t1, b, batch, Wf, bf):
    return pl.pallas_call(
        _pool_final_body,
        grid=(GRID,),
        in_specs=[
            pl.BlockSpec((BN, FH), lambda i: (i, 0)),
            pl.BlockSpec((BN, FH), lambda i: (i, 0)),
            pl.BlockSpec((1, HIDDEN), lambda i: (0, 0)),
            pl.BlockSpec((BN, 1), lambda i: (i, 0)),
            pl.BlockSpec((HIDDEN, OUT_DIM), lambda i: (0, 0)),
            pl.BlockSpec((1, OUT_DIM), lambda i: (0, 0)),
        ],
        out_specs=pl.BlockSpec((NUM_GRAPHS, OUT_DIM), lambda i: (0, 0)),
        out_shape=jax.ShapeDtypeStruct((NUM_GRAPHS, OUT_DIM), jnp.float32),
        scratch_shapes=[pltpu.VMEM((NUM_GRAPHS, HIDDEN), jnp.float32)],
    )(t0, t1, b[None, :], batch[:, None], Wf, bf[None, :])


_SC_MESH = dict(core_axis_name="c", subcore_axis_name="s",
                num_cores=NSC, num_subcores=NSUB)


def _emb_gather(emb_table, node_idx):
    mesh = plsc.VectorSubcoreMesh(**_SC_MESH)

    @functools.partial(
        pl.kernel, mesh=mesh,
        out_type=jax.ShapeDtypeStruct((N, FH), jnp.float32),
        scratch_types=[
            pltpu.VMEM((G,), jnp.int32),
            pltpu.VMEM((G, FH), jnp.float32),
            pltpu.SemaphoreType.DMA,
        ],
    )
    def k(emb_hbm, idx_hbm, out_hbm, idx_v, rows_v, sem):
        wid = lax.axis_index("s") * NSC + lax.axis_index("c")
        rlo = wid * 320
        rn = jnp.minimum(320, N - rlo)
        nq = rn // G

        def chunk(q, _):
            base = rlo + q * G
            pltpu.sync_copy(idx_hbm.at[pl.ds(base, G)], idx_v)
            pltpu.async_copy(emb_hbm.at[idx_v], rows_v, sem).wait()
            pltpu.sync_copy(rows_v, out_hbm.at[pl.ds(base, G)])
            return _

        lax.fori_loop(0, nq, chunk, None)

    return k(emb_table, node_idx)


def _sc_aggregate(H2, packed, a_src, a_dst, cvec, row_lo):
    mesh = plsc.VectorSubcoreMesh(**_SC_MESH)

    @functools.partial(
        pl.kernel, mesh=mesh,
        compiler_params=pltpu.CompilerParams(needs_layout_passes=False),
        out_type=jax.ShapeDtypeStruct((NSC, NACC, FH), jnp.float32),
        scratch_types=[
            pltpu.VMEM((N,), jnp.float32),
            pltpu.VMEM((NDT,), jnp.float32),
            pltpu.VMEM((16,), jnp.float32),
            pltpu.VMEM((EB,), jnp.int32),
            pltpu.VMEM((EB,), jnp.int32),
            pltpu.VMEM((NQ, G), jnp.int32),
            pltpu.VMEM((NQ, G), jnp.float32),
            pltpu.VMEM((2, G, FH), jnp.float32),
            pltpu.VMEM((2, G, FH), jnp.float32),
            pltpu.VMEM_SHARED((NACC, FH), jnp.float32),
            pltpu.VMEM_SHARED((NACC,), jnp.float32),
            pltpu.SemaphoreType.DMA,
            pltpu.SemaphoreType.DMA,
        ],
    )
    def k(h2_hbm, pk_hbm, asrc_hbm, adst_hbm, c_hbm, out_hbm,
          asrc_v, adst_v, c_v, p_v, s_v, d_v, e_v, rows_v, stag_v,
          acc_sh, den_sh, sem_g, sem_s):
        lane = lax.iota(jnp.int32, 16)
        c = lax.axis_index("c")
        s = lax.axis_index("s")
        rlo = s * RW
        nz = RW // G

        pltpu.sync_copy(asrc_hbm, asrc_v)
        pltpu.sync_copy(adst_hbm, adst_v)
        pltpu.sync_copy(c_hbm, c_v)
        cv = c_v[...]

        def zrow(j, _):
            for kk in range(FH // 16):
                stag_v[0, j, pl.ds(kk * 16, 16)] = jnp.zeros((16,), jnp.float32)
            return _

        lax.fori_loop(0, G, zrow, None)

        def zacc(i, _):
            pltpu.sync_copy(stag_v.at[0], acc_sh.at[pl.ds(rlo + i * G, G)])
            pltpu.sync_copy(stag_v.at[0, 0, pl.ds(0, G)],
                            den_sh.at[pl.ds(rlo + i * G, G)])
            return _

        lax.fori_loop(0, nz, zacc, None)
        plsc.subcore_barrier()

        def ealpha(sv, dv):
            al = plsc.load_gather(asrc_v, [sv]) + plsc.load_gather(adst_v, [dv])
            al = jnp.where(al >= 0, al, jnp.float32(0.2) * al)
            return jnp.exp(al - cv)

        def scale_chunk(buf, q_ev):
            def sgroup(jj, _):
                ev16 = e_v[q_ev, pl.ds(jj * 16, 16)]
                for jl in range(16):
                    j = jj * 16 + jl
                    e = ev16[jl]
                    for kk in range(FH // 16):
                        stag_v[buf, j, pl.ds(kk * 16, 16)] = (
                            rows_v[buf, j, pl.ds(kk * 16, 16)] * e)
                return _

            lax.fori_loop(0, G // 16, sgroup, None)

        def gather_issue(q, pw):
            pltpu.async_copy(h2_hbm.at[c].at[s_v.at[pl.ds(q * G, G)]],
                             rows_v.at[pw], sem_g)

        def gather_wait(q, pw):
            pltpu.make_async_copy(
                h2_hbm.at[c].at[s_v.at[pl.ds(q * G, G)]],
                rows_v.at[pw], sem_g).wait()

        def scatter_issue(q, pw):
            pltpu.async_copy(stag_v.at[pw], acc_sh.at[d_v.at[q]], sem_s,
                             add=True)
            pltpu.async_copy(e_v.at[q], den_sh.at[d_v.at[q]], sem_s,
                             add=True)

        def scatter_wait(q, pw):
            pltpu.make_async_copy(stag_v.at[pw], acc_sh.at[d_v.at[q]],
                                  sem_s).wait()
            pltpu.make_async_copy(e_v.at[q], den_sh.at[d_v.at[q]],
                                  sem_s).wait()

        def block(b, _):
            pltpu.sync_copy(pk_hbm.at[s, b], p_v)

            def scan(r, _):
                off = r * 16
                row = r // (G // 16)
                col = (r % (G // 16)) * 16
                pv = p_v[pl.ds(off, 16)]
                sv = lax.bitwise_and(pv, jnp.int32(0xFFFF))
                dv = lax.shift_right_logical(pv, 16)
                dloc = dv - jnp.int32(row_lo)
                inr = (dloc >= 0) & (dloc < NH)
                ev = ealpha(sv, dv)
                s_v[pl.ds(off, 16)] = sv
                d_v[row, pl.ds(col, 16)] = jnp.where(inr, dloc, jnp.int32(NH))
                e_v[row, pl.ds(col, 16)] = jnp.where(inr, ev, jnp.float32(0.0))
                return _

            lax.fori_loop(0, EB // 16, scan, None)

            gather_issue(0, 0)

            def chunk(q, _):
                pw = lax.rem(q, 2)
                gather_wait(q, pw)

                @pl.when(q < NQ - 1)
                def _():
                    gather_issue(q + 1, 1 - pw)

                @pl.when(q >= 2)
                def _():
                    scatter_wait(q - 2, pw)

                scale_chunk(pw, q)
                scatter_issue(q, pw)
                return _

            lax.fori_loop(0, NQ, chunk, None)
            scatter_wait(NQ - 2, (NQ - 2) % 2)
            scatter_wait(NQ - 1, (NQ - 1) % 2)
            return _

        lax.fori_loop(0, NB, block, None)

        plsc.subcore_barrier()

        def readout(i, _):
            r0 = rlo + i * G
            pltpu.sync_copy(acc_sh.at[pl.ds(r0, G)], rows_v.at[0])
            pltpu.sync_copy(den_sh.at[pl.ds(r0, G)], e_v.at[0])

            def recip(jj, _):
                dv = e_v[0, pl.ds(jj * 16, 16)]
                e_v[0, pl.ds(jj * 16, 16)] = (
                    jnp.float32(1.0) / (dv + jnp.float32(1e-16)))
                return _

            lax.fori_loop(0, G // 16, recip, None)
            scale_chunk(0, 0)
            pltpu.sync_copy(stag_v.at[0], out_hbm.at[c, pl.ds(r0, G)])
            return _

        lax.fori_loop(0, nz, readout, None)

    return k(H2, packed, a_src, a_dst, cvec)


def _sc_layer(H2, packed, a_src, a_dst_pad, cvec):
    lo = _sc_aggregate(H2, packed, a_src, a_dst_pad, cvec, 0)
    hi = _sc_aggregate(H2, packed, a_src, a_dst_pad, cvec, NH)
    t0 = jnp.concatenate([lo[0, :NH], hi[0, :NH]], axis=0)
    t1 = jnp.concatenate([lo[1, :NH], hi[1, :NH]], axis=0)
    return t0, t1


def kernel(x, edge_attr, emb_table, W1, as1, ad1, b1, W2, as2, ad2, b2,
           W3, as3, ad3, b3, Wf, bf, edge_index, batch):
    src = edge_index[0]
    dst = edge_index[1]
    loop = jnp.arange(N, dtype=jnp.int32)
    dummy = jnp.full((ET - E - N,), N * 65536, jnp.int32)
    packed = jnp.concatenate(
        [src + dst * 65536, loop * 65537, dummy]).reshape(NSUB, NB, EB)
    node_idx = x[:, -1].astype(jnp.int32)
    emb = _emb_gather(emb_table, node_idx)
    t0, t1 = x[:, :FH], emb
    zero_b = jnp.zeros((HIDDEN,), jnp.float32)

    def layer(t0, t1, b, W, att_s, att_d, first):
        H2, a_s, a_d, cmax = _mm_att(t0, t1, b, W, att_s, att_d, first)
        cvec = jnp.full((16,), cmax[0, 0] + cmax[0, 1], jnp.float32)
        a_d_pad = jnp.concatenate(
            [a_d[:, 0], jnp.zeros((NDT - N,), jnp.float32)])
        return _sc_layer(H2, packed, a_s[:, 0], a_d_pad, cvec)

    t0, t1 = layer(t0, t1, zero_b, W1, as1, ad1, first=True)
    t0, t1 = layer(t0, t1, b1, W2, as2, ad2, first=False)
    t0, t1 = layer(t0, t1, b2, W3, as3, ad3, first=False)
    return _pool_final(t0, t1, b3, batch, Wf, bf)


# speedup vs baseline: 1.1952x; 1.1952x over previous
import functools

import jax
import jax.numpy as jnp
from jax import lax
from jax.experimental import pallas as pl
from jax.experimental.pallas import tpu as pltpu
from jax.experimental.pallas import tpu_sc as plsc

N = 10000
E = 320000
HIDDEN = 256
FH = 128
ACCW = FH
NUM_GRAPHS = 64
OUT_DIM = 64
BN = 1000
GRID = N // BN

NSC = 2
NSUB = 16
ET = 332800
EB = 2080
NB = 10
ES = NB * EB
G = 80
NQ = EB // G
NH = 5000
NACC = 5120
RW = 320
NDT = 10016


def _gelu_exact(x):
    return 0.5 * x * (1.0 + jax.lax.erf(x * jnp.float32(0.7071067811865476)))


def _mm_att_body(first, t0_ref, t1_ref, b_ref, W_ref, as_ref, ad_ref,
                 H_ref, asrc_ref, adst_ref, cmax_ref):
    i = pl.program_id(0)
    t = jnp.concatenate([t0_ref[...], t1_ref[...]], axis=1)
    if not first:
        t = _gelu_exact(t + b_ref[...])
    H = jax.lax.dot_general(t, W_ref[...], (((1,), (0,)), ((), ())),
                            preferred_element_type=jnp.float32,
                            precision=jax.lax.Precision.HIGHEST)
    H_ref[0] = H[:, :FH]
    H_ref[1] = H[:, FH:]
    asrc = jnp.sum(H * as_ref[...], axis=1, keepdims=True)
    adst = jnp.sum(H * ad_ref[...], axis=1, keepdims=True)
    asrc_ref[...] = asrc
    adst_ref[...] = adst
    m = jnp.concatenate([jnp.max(asrc, axis=0), jnp.max(adst, axis=0)])[None, :]

    @pl.when(i == 0)
    def _():
        cmax_ref[...] = m

    @pl.when(i > 0)
    def _():
        cmax_ref[...] = jnp.maximum(cmax_ref[...], m)


def _mm_att(t0, t1, b, W, att_s, att_d, first):
    body = functools.partial(_mm_att_body, first)
    return pl.pallas_call(
        body,
        grid=(GRID,),
        in_specs=[
            pl.BlockSpec((BN, FH), lambda i: (i, 0)),
            pl.BlockSpec((BN, FH), lambda i: (i, 0)),
            pl.BlockSpec((1, HIDDEN), lambda i: (0, 0)),
            pl.BlockSpec((HIDDEN, HIDDEN), lambda i: (0, 0)),
            pl.BlockSpec((1, HIDDEN), lambda i: (0, 0)),
            pl.BlockSpec((1, HIDDEN), lambda i: (0, 0)),
        ],
        out_specs=[
            pl.BlockSpec((NSC, BN, FH), lambda i: (0, i, 0)),
            pl.BlockSpec((BN, 1), lambda i: (i, 0)),
            pl.BlockSpec((BN, 1), lambda i: (i, 0)),
            pl.BlockSpec((1, 2), lambda i: (0, 0)),
        ],
        out_shape=[
            jax.ShapeDtypeStruct((NSC, N, FH), jnp.float32),
            jax.ShapeDtypeStruct((N, 1), jnp.float32),
            jax.ShapeDtypeStruct((N, 1), jnp.float32),
            jax.ShapeDtypeStruct((1, 2), jnp.float32),
        ],
    )(t0, t1, b[None, :], W, att_s[None, :], att_d[None, :])


def _pool_final_body(t0_ref, t1_ref, b_ref, batch_ref, Wf_ref, bf_ref,
                     out_ref, acc_ref):
    i = pl.program_id(0)
    t = jnp.concatenate([t0_ref[...], t1_ref[...]], axis=1)
    t = _gelu_exact(t + b_ref[...])
    g = batch_ref[...]
    onehot = (g == jax.lax.broadcasted_iota(jnp.int32, (1, NUM_GRAPHS), 1)
              ).astype(jnp.float32)
    part = jax.lax.dot_general(onehot, t, (((0,), (0,)), ((), ())),
                               preferred_element_type=jnp.float32,
                               precision=jax.lax.Precision.HIGHEST)

    @pl.when(i == 0)
    def _():
        acc_ref[...] = jnp.zeros_like(acc_ref)

    acc_ref[...] += part

    @pl.when(i == GRID - 1)
    def _():
        o = jax.lax.dot_general(acc_ref[...], Wf_ref[...], (((1,), (0,)), ((), ())),
                                preferred_element_type=jnp.float32,
                                precision=jax.lax.Precision.HIGHEST) + bf_ref[...]
        out_ref[...] = jnp.where(o >= 0, o, 0.01 * o)


def _pool_final(t0, t1, b, batch, Wf, bf):
    return pl.pallas_call(
        _pool_final_body,
        grid=(GRID,),
        in_specs=[
            pl.BlockSpec((BN, FH), lambda i: (i, 0)),
            pl.BlockSpec((BN, FH), lambda i: (i, 0)),
            pl.BlockSpec((1, HIDDEN), lambda i: (0, 0)),
            pl.BlockSpec((BN, 1), lambda i: (i, 0)),
            pl.BlockSpec((HIDDEN, OUT_DIM), lambda i: (0, 0)),
            pl.BlockSpec((1, OUT_DIM), lambda i: (0, 0)),
        ],
        out_specs=pl.BlockSpec((NUM_GRAPHS, OUT_DIM), lambda i: (0, 0)),
        out_shape=jax.ShapeDtypeStruct((NUM_GRAPHS, OUT_DIM), jnp.float32),
        scratch_shapes=[pltpu.VMEM((NUM_GRAPHS, HIDDEN), jnp.float32)],
    )(t0, t1, b[None, :], batch[:, None], Wf, bf[None, :])


_SC_MESH = dict(core_axis_name="c", subcore_axis_name="s",
                num_cores=NSC, num_subcores=NSUB)


def _emb_gather(emb_table, node_idx):
    mesh = plsc.VectorSubcoreMesh(**_SC_MESH)

    @functools.partial(
        pl.kernel, mesh=mesh,
        out_type=jax.ShapeDtypeStruct((N, FH), jnp.float32),
        scratch_types=[
            pltpu.VMEM((G,), jnp.int32),
            pltpu.VMEM((G, FH), jnp.float32),
            pltpu.SemaphoreType.DMA,
        ],
    )
    def k(emb_hbm, idx_hbm, out_hbm, idx_v, rows_v, sem):
        wid = lax.axis_index("s") * NSC + lax.axis_index("c")
        rlo = wid * 320
        rn = jnp.minimum(320, N - rlo)
        nq = rn // G

        def chunk(q, _):
            base = rlo + q * G
            pltpu.sync_copy(idx_hbm.at[pl.ds(base, G)], idx_v)
            pltpu.async_copy(emb_hbm.at[idx_v], rows_v, sem).wait()
            pltpu.sync_copy(rows_v, out_hbm.at[pl.ds(base, G)])
            return _

        lax.fori_loop(0, nq, chunk, None)

    return k(emb_table, node_idx)


def _sc_aggregate(H2, packed, a_src, a_dst, cvec, row_lo):
    mesh = plsc.VectorSubcoreMesh(**_SC_MESH)

    @functools.partial(
        pl.kernel, mesh=mesh,
        compiler_params=pltpu.CompilerParams(needs_layout_passes=False),
        out_type=jax.ShapeDtypeStruct((NSC, NACC, FH), jnp.float32),
        scratch_types=[
            pltpu.VMEM((N,), jnp.float32),
            pltpu.VMEM((NDT,), jnp.float32),
            pltpu.VMEM((16,), jnp.float32),
            pltpu.VMEM((EB,), jnp.int32),
            pltpu.VMEM((EB,), jnp.int32),
            pltpu.VMEM((NQ, G), jnp.int32),
            pltpu.VMEM((NQ, G), jnp.float32),
            pltpu.VMEM((2, G, FH), jnp.float32),
            pltpu.VMEM((2, G, FH), jnp.float32),
            pltpu.VMEM_SHARED((NACC, FH), jnp.float32),
            pltpu.VMEM_SHARED((NACC,), jnp.float32),
            pltpu.SemaphoreType.DMA,
            pltpu.SemaphoreType.DMA,
        ],
    )
    def k(h2_hbm, pk_hbm, asrc_hbm, adst_hbm, c_hbm, out_hbm,
          asrc_v, adst_v, c_v, p_v, s_v, d_v, e_v, rows_v, stag_v,
          acc_sh, den_sh, sem_g, sem_s):
        lane = lax.iota(jnp.int32, 16)
        c = lax.axis_index("c")
        s = lax.axis_index("s")
        rlo = s * RW
        nz = RW // G

        pltpu.sync_copy(asrc_hbm, asrc_v)
        pltpu.sync_copy(adst_hbm, adst_v)
        pltpu.sync_copy(c_hbm, c_v)
        cv = c_v[...]

        def zrow(j, _):
            for kk in range(FH // 16):
                stag_v[0, j, pl.ds(kk * 16, 16)] = jnp.zeros((16,), jnp.float32)
            return _

        lax.fori_loop(0, G, zrow, None)

        def zacc(i, _):
            pltpu.sync_copy(stag_v.at[0], acc_sh.at[pl.ds(rlo + i * G, G)])
            pltpu.sync_copy(stag_v.at[0, 0, pl.ds(0, G)],
                            den_sh.at[pl.ds(rlo + i * G, G)])
            return _

        lax.fori_loop(0, nz, zacc, None)
        plsc.subcore_barrier()

        def ealpha(sv, dv):
            al = plsc.load_gather(asrc_v, [sv]) + plsc.load_gather(adst_v, [dv])
            al = jnp.where(al >= 0, al, jnp.float32(0.2) * al)
            return jnp.exp(al - cv)

        def scale_chunk(buf, q_ev):
            def sgroup(jj, _):
                ev16 = e_v[q_ev, pl.ds(jj * 16, 16)]
                for jl in range(16):
                    j = jj * 16 + jl
                    e = ev16[jl]
                    for kk in range(FH // 16):
                        stag_v[buf, j, pl.ds(kk * 16, 16)] = (
                            rows_v[buf, j, pl.ds(kk * 16, 16)] * e)
                return _

            lax.fori_loop(0, G // 16, sgroup, None)

        def gather_issue(q, pw):
            return pltpu.async_copy(
                h2_hbm.at[c].at[s_v.at[pl.ds(q * G, G)]],
                rows_v.at[pw], sem_g)

        def scatter_issue(q, pw):
            return (pltpu.async_copy(stag_v.at[pw], acc_sh.at[d_v.at[q]],
                                     sem_s, add=True),
                    pltpu.async_copy(e_v.at[q], den_sh.at[d_v.at[q]],
                                     sem_s, add=True))

        def block(b, _):
            pltpu.sync_copy(pk_hbm.at[s, b], p_v)

            def scan(r, _):
                off = r * 16
                row = r // (G // 16)
                col = (r % (G // 16)) * 16
                pv = p_v[pl.ds(off, 16)]
                sv = lax.bitwise_and(pv, jnp.int32(0xFFFF))
                dv = lax.shift_right_logical(pv, 16)
                dloc = dv - jnp.int32(row_lo)
                inr = (dloc >= 0) & (dloc < NH)
                ev = ealpha(sv, dv)
                s_v[pl.ds(off, 16)] = sv
                d_v[row, pl.ds(col, 16)] = jnp.where(inr, dloc, jnp.int32(NH))
                e_v[row, pl.ds(col, 16)] = jnp.where(inr, ev, jnp.float32(0.0))
                return _

            lax.fori_loop(0, EB // 16, scan, None)

            def chunk(q, _):
                gather_issue(q, 0).wait()
                scale_chunk(0, q)
                for dsc in scatter_issue(q, 0):
                    dsc.wait()
                return _

            lax.fori_loop(0, NQ, chunk, None)
            return _

        lax.fori_loop(0, NB, block, None)

        plsc.subcore_barrier()

        def readout(i, _):
            r0 = rlo + i * G
            pltpu.sync_copy(acc_sh.at[pl.ds(r0, G)], rows_v.at[0])
            pltpu.sync_copy(den_sh.at[pl.ds(r0, G)], e_v.at[0])

            def recip(jj, _):
                dv = e_v[0, pl.ds(jj * 16, 16)]
                e_v[0, pl.ds(jj * 16, 16)] = (
                    jnp.float32(1.0) / (dv + jnp.float32(1e-16)))
                return _

            lax.fori_loop(0, G // 16, recip, None)
            scale_chunk(0, 0)
            pltpu.sync_copy(stag_v.at[0], out_hbm.at[c, pl.ds(r0, G)])
            return _

        lax.fori_loop(0, nz, readout, None)

    return k(H2, packed, a_src, a_dst, cvec)


def _sc_layer(H2, packed, a_src, a_dst_pad, cvec):
    lo = _sc_aggregate(H2, packed, a_src, a_dst_pad, cvec, 0)
    hi = _sc_aggregate(H2, packed, a_src, a_dst_pad, cvec, NH)
    t0 = jnp.concatenate([lo[0, :NH], hi[0, :NH]], axis=0)
    t1 = jnp.concatenate([lo[1, :NH], hi[1, :NH]], axis=0)
    return t0, t1


def kernel(x, edge_attr, emb_table, W1, as1, ad1, b1, W2, as2, ad2, b2,
           W3, as3, ad3, b3, Wf, bf, edge_index, batch):
    src = edge_index[0]
    dst = edge_index[1]
    loop = jnp.arange(N, dtype=jnp.int32)
    dummy = jnp.full((ET - E - N,), N * 65536, jnp.int32)
    packed = jnp.concatenate(
        [src + dst * 65536, loop * 65537, dummy]).reshape(NSUB, NB, EB)
    node_idx = x[:, -1].astype(jnp.int32)
    emb = _emb_gather(emb_table, node_idx)
    t0, t1 = x[:, :FH], emb
    zero_b = jnp.zeros((HIDDEN,), jnp.float32)

    def layer(t0, t1, b, W, att_s, att_d, first):
        H2, a_s, a_d, cmax = _mm_att(t0, t1, b, W, att_s, att_d, first)
        cvec = jnp.full((16,), cmax[0, 0] + cmax[0, 1], jnp.float32)
        a_d_pad = jnp.concatenate(
            [a_d[:, 0], jnp.zeros((NDT - N,), jnp.float32)])
        return _sc_layer(H2, packed, a_s[:, 0], a_d_pad, cvec)

    t0, t1 = layer(t0, t1, zero_b, W1, as1, ad1, first=True)
    t0, t1 = layer(t0, t1, b1, W2, as2, ad2, first=False)
    t0, t1 = layer(t0, t1, b2, W3, as3, ad3, first=False)
    return _pool_final(t0, t1, b3, batch, Wf, bf)
